# baseline (speedup 1.0000x reference)
import jax
import jax.numpy as jnp
from jax.experimental import pallas as pl

_N_TYPES = 100000
_SEQ_LEN = 200
_DBLK = 16384


def _bow_block_kernel(tok_ref, val_ref, col_ref, out_ref):
    j = pl.program_id(0)
    tokb = jnp.broadcast_to(tok_ref[:, 0:1], (_SEQ_LEN, _DBLK))
    valb = jnp.broadcast_to(val_ref[:, 0:1], (_SEQ_LEN, _DBLK))
    colb = jnp.tile(col_ref[0:8, :], (_SEQ_LEN // 8, 1))
    mask = tokb - j * _DBLK == colb
    out_ref[:, :] = jnp.where(mask, valb, 0.0)


def kernel(tokens, vals):
    tok2 = jnp.broadcast_to(tokens.astype(jnp.int32)[:, None, None], (_SEQ_LEN, 1, 128))
    val2 = jnp.broadcast_to(vals[:, None, None], (_SEQ_LEN, 1, 128))
    col2 = jnp.arange(_DBLK, dtype=jnp.int32)[None, None, :] * jnp.ones((8, 1, 1), jnp.int32)
    grid = (pl.cdiv(_N_TYPES, _DBLK),)
    out = pl.pallas_call(
        _bow_block_kernel,
        grid=grid,
        in_specs=[
            pl.BlockSpec((_SEQ_LEN, None, 128), lambda j: (0, 0, 0)),
            pl.BlockSpec((_SEQ_LEN, None, 128), lambda j: (0, 0, 0)),
            pl.BlockSpec((8, None, _DBLK), lambda j: (0, 0, 0)),
        ],
        out_specs=pl.BlockSpec((_SEQ_LEN, None, _DBLK), lambda j: (0, 0, j)),
        out_shape=jax.ShapeDtypeStruct((_SEQ_LEN, 1, _N_TYPES), jnp.float32),
    )(tok2, val2, col2)
    return out


# speedup vs baseline: 1.0274x; 1.0274x over previous
import jax
import jax.numpy as jnp
from jax.experimental import pallas as pl

_N_TYPES = 100000
_SEQ_LEN = 200
_DBLK = 12288


def _bow_block_kernel(tok_ref, val_ref, col_ref, out_ref):
    j = pl.program_id(0)
    tokb = jnp.broadcast_to(tok_ref[:, 0:1], (_SEQ_LEN, _DBLK))
    valb = jnp.broadcast_to(val_ref[:, 0:1], (_SEQ_LEN, _DBLK))
    colb = jnp.tile(col_ref[0:8, :], (_SEQ_LEN // 8, 1))
    mask = tokb - j * _DBLK == colb
    out_ref[:, :] = jnp.where(mask, valb, 0.0)


def kernel(tokens, vals):
    tok2 = jnp.broadcast_to(tokens.astype(jnp.int32)[:, None, None], (_SEQ_LEN, 1, 128))
    val2 = jnp.broadcast_to(vals[:, None, None], (_SEQ_LEN, 1, 128))
    col2 = jnp.arange(_DBLK, dtype=jnp.int32)[None, None, :] * jnp.ones((8, 1, 1), jnp.int32)
    grid = (pl.cdiv(_N_TYPES, _DBLK),)
    out = pl.pallas_call(
        _bow_block_kernel,
        grid=grid,
        in_specs=[
            pl.BlockSpec((_SEQ_LEN, None, 128), lambda j: (0, 0, 0)),
            pl.BlockSpec((_SEQ_LEN, None, 128), lambda j: (0, 0, 0)),
            pl.BlockSpec((8, None, _DBLK), lambda j: (0, 0, 0)),
        ],
        out_specs=pl.BlockSpec((_SEQ_LEN, None, _DBLK), lambda j: (0, 0, j)),
        out_shape=jax.ShapeDtypeStruct((_SEQ_LEN, 1, _N_TYPES), jnp.float32),
    )(tok2, val2, col2)
    return out
